# baseline (speedup 1.0000x reference)
import jax
import jax.numpy as jnp
from jax import lax
from jax.experimental import pallas as pl
from jax.experimental.pallas import tpu as pltpu
from jax.experimental.pallas import tpu_sc as plsc

_N = 10000
_E = 320000
_NC, _NS = 2, 16
_NW = _NC * _NS
_CHUNK = 128
_NCHG = 2560
_E_PAD = _NCHG * _CHUNK
_NBUF = 8
_NB2 = 2 * _NBUF
_A = 10240
_RPT = _A // _NS
_ROW_BLK = 400
_GRID = _N // _ROW_BLK


def _sc_agg_call(y, sd, d, with_counts):
    mesh = plsc.VectorSubcoreMesh(core_axis_name="c", subcore_axis_name="s")
    nch = _NCHG // _NW

    out_type = [jax.ShapeDtypeStruct((_NC, _A, d), jnp.float32)]
    scratch = [
        pltpu.VMEM((2, _CHUNK), jnp.int32),
        pltpu.VMEM((_CHUNK, d), jnp.float32),
        pltpu.VMEM_SHARED((_A, d), jnp.float32),
        pltpu.SemaphoreType.DMA,
    ]
    if with_counts:
        out_type.append(jax.ShapeDtypeStruct((_NC, _A, 16), jnp.float32))
        scratch += [
            pltpu.VMEM((_CHUNK, 16), jnp.float32),
            pltpu.VMEM_SHARED((_A, 16), jnp.float32),
        ]

    def body(y_hbm, sd_hbm, zero_hbm, zero16_hbm, ones_hbm,
             *out_and_scratch):
        if with_counts:
            (out_hbm, cnt_hbm, sd_v, rows_v, acc_sh, sem,
             ones_v, cnt_sh) = out_and_scratch
        else:
            out_hbm, sd_v, rows_v, acc_sh, sem = out_and_scratch

        c = lax.axis_index("c")
        s = lax.axis_index("s")
        wid = c * _NS + s

        pltpu.sync_copy(zero_hbm, acc_sh.at[pl.ds(s * _RPT, _RPT)])
        if with_counts:
            pltpu.sync_copy(zero16_hbm, cnt_sh.at[pl.ds(s * _RPT, _RPT)])
            pltpu.sync_copy(ones_hbm, ones_v)
        plsc.subcore_barrier()

        base = wid * nch

        def step(i, carry):
            pltpu.sync_copy(sd_hbm.at[base + i], sd_v)
            pltpu.async_copy(y_hbm.at[sd_v.at[0]], rows_v, sem).wait()
            pltpu.sync_copy(rows_v, acc_sh.at[sd_v.at[1]], add=True)
            if with_counts:
                pltpu.sync_copy(ones_v, cnt_sh.at[sd_v.at[1]], add=True)
            return carry

        lax.fori_loop(0, nch, step, 0)
        plsc.subcore_barrier()

        pltpu.sync_copy(acc_sh.at[pl.ds(s * _RPT, _RPT)],
                        out_hbm.at[c, pl.ds(s * _RPT, _RPT)])
        if with_counts:
            pltpu.sync_copy(cnt_sh.at[pl.ds(s * _RPT, _RPT)],
                            cnt_hbm.at[c, pl.ds(s * _RPT, _RPT)])

    zero = jnp.zeros((_RPT, d), jnp.float32)
    zero16 = jnp.zeros((_RPT, 16), jnp.float32)
    ones = jnp.ones((_CHUNK, 16), jnp.float32)
    fn = pl.kernel(
        body, out_type=out_type, mesh=mesh, scratch_types=scratch,
        compiler_params=pltpu.CompilerParams(use_tc_tiling_on_sc=False))
    return fn(y, sd, zero, zero16, ones)


def _lin2_body(h_ref, wl_ref, wr_ref, b_ref, y_ref, z_ref):
    h = h_ref[...]
    y_ref[...] = jnp.dot(h, wl_ref[...], preferred_element_type=jnp.float32)
    z_ref[...] = (jnp.dot(h, wr_ref[...], preferred_element_type=jnp.float32)
                  + b_ref[...])


def _tc_lin2(h, wl, wr, b):
    din, dout = wl.shape
    return pl.pallas_call(
        _lin2_body,
        grid=(_GRID,),
        in_specs=[
            pl.BlockSpec((_ROW_BLK, din), lambda i: (i, 0)),
            pl.BlockSpec((din, dout), lambda i: (0, 0)),
            pl.BlockSpec((din, dout), lambda i: (0, 0)),
            pl.BlockSpec((1, dout), lambda i: (0, 0)),
        ],
        out_specs=[
            pl.BlockSpec((_ROW_BLK, dout), lambda i: (i, 0)),
            pl.BlockSpec((_ROW_BLK, dout), lambda i: (i, 0)),
        ],
        out_shape=[
            jax.ShapeDtypeStruct((_N, dout), jnp.float32),
            jax.ShapeDtypeStruct((_N, dout), jnp.float32),
        ],
    )(h, wl, wr, b.reshape(1, dout))


def _combine_h(p_ref, cnt_ref, z_ref):
    cnt = cnt_ref[0, :, 0:1] + cnt_ref[1, :, 0:1]
    scale = 1.0 / jnp.maximum(cnt, 1.0)
    return (p_ref[0] + p_ref[1]) * scale + z_ref[...]


def _comb_body(p_ref, cnt_ref, z_ref, wl_ref, wr_ref, b_ref, y_ref, z2_ref):
    h = jnp.maximum(_combine_h(p_ref, cnt_ref, z_ref), 0.0)
    y_ref[...] = jnp.dot(h, wl_ref[...], preferred_element_type=jnp.float32)
    z2_ref[...] = (jnp.dot(h, wr_ref[...], preferred_element_type=jnp.float32)
                   + b_ref[...])


def _tc_combine(p, cntp, z, wl, wr, b):
    dp = p.shape[-1]
    din, dout = wl.shape
    return pl.pallas_call(
        _comb_body,
        grid=(_GRID,),
        in_specs=[
            pl.BlockSpec((2, _ROW_BLK, dp), lambda i: (0, i, 0)),
            pl.BlockSpec((2, _ROW_BLK, 16), lambda i: (0, i, 0)),
            pl.BlockSpec((_ROW_BLK, dp), lambda i: (i, 0)),
            pl.BlockSpec((din, dout), lambda i: (0, 0)),
            pl.BlockSpec((din, dout), lambda i: (0, 0)),
            pl.BlockSpec((1, dout), lambda i: (0, 0)),
        ],
        out_specs=[
            pl.BlockSpec((_ROW_BLK, dout), lambda i: (i, 0)),
            pl.BlockSpec((_ROW_BLK, dout), lambda i: (i, 0)),
        ],
        out_shape=[
            jax.ShapeDtypeStruct((_N, dout), jnp.float32),
            jax.ShapeDtypeStruct((_N, dout), jnp.float32),
        ],
    )(p, cntp, z, wl, wr, b.reshape(1, dout))


def _final_body(p_ref, cnt_ref, z_ref, o_ref):
    h = _combine_h(p_ref, cnt_ref, z_ref)
    m = jnp.max(h, axis=1, keepdims=True)
    e = jnp.exp(h - m)
    o_ref[...] = h - (jnp.log(jnp.sum(e, axis=1, keepdims=True)) + m)


def _tc_final(p, cntp, z):
    dp = p.shape[-1]
    return pl.pallas_call(
        _final_body,
        grid=(_GRID,),
        in_specs=[
            pl.BlockSpec((2, _ROW_BLK, dp), lambda i: (0, i, 0)),
            pl.BlockSpec((2, _ROW_BLK, 16), lambda i: (0, i, 0)),
            pl.BlockSpec((_ROW_BLK, dp), lambda i: (i, 0)),
        ],
        out_specs=pl.BlockSpec((_ROW_BLK, dp), lambda i: (i, 0)),
        out_shape=jax.ShapeDtypeStruct((_N, dp), jnp.float32),
    )(p, cntp, z)


def kernel(x, edge_index, Wl1, bl1, Wr1, Wl2, bl2, Wr2,
           Wl3, bl3, Wr3, Wl4, bl4, Wr4):
    pad = _E_PAD - _E
    src = jnp.concatenate(
        [edge_index[0], jnp.zeros((pad,), jnp.int32)]).reshape(-1, _CHUNK)
    dst = jnp.concatenate(
        [edge_index[1], jnp.full((pad,), _N, jnp.int32)]).reshape(-1, _CHUNK)
    sd = jnp.stack([src, dst], axis=1)

    y1, z1 = _tc_lin2(x, Wl1, Wr1, bl1)
    p1, cp = _sc_agg_call(y1, sd, 128, with_counts=True)
    p1, cp = p1[:, :_N], cp[:, :_N]

    y2, z2 = _tc_combine(p1, cp, z1, Wl2, Wr2, bl2)
    p2 = _sc_agg_call(y2, sd, 128, with_counts=False)[0][:, :_N]

    y3, z3 = _tc_combine(p2, cp, z2, Wl3, Wr3, bl3)
    p3 = _sc_agg_call(y3, sd, 64, with_counts=False)[0][:, :_N]

    y4, z4 = _tc_combine(p3, cp, z3, Wl4, Wr4, bl4)
    p4 = _sc_agg_call(y4, sd, 64, with_counts=False)[0][:, :_N]

    return _tc_final(p4, cp, z4)


# speedup vs baseline: 1.7972x; 1.7972x over previous
import jax
import jax.numpy as jnp
from jax import lax
from jax.experimental import pallas as pl
from jax.experimental.pallas import tpu as pltpu
from jax.experimental.pallas import tpu_sc as plsc

_N = 10000
_E = 320000
_NC, _NS = 2, 16
_NW = _NC * _NS
_CHUNK = 128
_NCHUNKS = _E // _CHUNK
_A = 10240
_RPT = _A // _NS
_ROW_BLK = 400
_GRID = _N // _ROW_BLK


def _sc_agg_call(y, src, dst, d, with_counts):
    mesh = plsc.VectorSubcoreMesh(core_axis_name="c", subcore_axis_name="s")

    out_type = [jax.ShapeDtypeStruct((_NC, _A, d), jnp.float32)]
    scratch = [
        pltpu.VMEM((_CHUNK,), jnp.int32),
        pltpu.VMEM((_CHUNK,), jnp.int32),
        pltpu.VMEM((_CHUNK, d), jnp.float32),
        pltpu.VMEM_SHARED((_A, d), jnp.float32),
        pltpu.SemaphoreType.DMA,
    ]
    if with_counts:
        out_type.append(jax.ShapeDtypeStruct((_NC, _A, 16), jnp.float32))
        scratch += [
            pltpu.VMEM((_CHUNK, 16), jnp.float32),
            pltpu.VMEM_SHARED((_A, 16), jnp.float32),
        ]

    def body(y_hbm, src_hbm, dst_hbm, zero_hbm, zero16_hbm, ones_hbm,
             *out_and_scratch):
        if with_counts:
            (out_hbm, cnt_hbm, src_v, dst_v, rows_v, acc_sh, sem,
             ones_v, cnt_sh) = out_and_scratch
        else:
            out_hbm, src_v, dst_v, rows_v, acc_sh, sem = out_and_scratch

        c = lax.axis_index("c")
        s = lax.axis_index("s")
        wid = c * _NS + s

        pltpu.sync_copy(zero_hbm, acc_sh.at[pl.ds(s * _RPT, _RPT)])
        if with_counts:
            pltpu.sync_copy(zero16_hbm, cnt_sh.at[pl.ds(s * _RPT, _RPT)])
            pltpu.sync_copy(ones_hbm, ones_v)
        plsc.subcore_barrier()

        lo = wid * _NCHUNKS // _NW
        hi = (wid + 1) * _NCHUNKS // _NW

        def step(i, carry):
            off = i * _CHUNK
            pltpu.sync_copy(src_hbm.at[pl.ds(off, _CHUNK)], src_v)
            pltpu.sync_copy(dst_hbm.at[pl.ds(off, _CHUNK)], dst_v)
            pltpu.async_copy(y_hbm.at[src_v], rows_v, sem).wait()
            pltpu.sync_copy(rows_v, acc_sh.at[dst_v], add=True)
            if with_counts:
                pltpu.sync_copy(ones_v, cnt_sh.at[dst_v], add=True)
            return carry

        lax.fori_loop(lo, hi, step, 0)
        plsc.subcore_barrier()

        pltpu.sync_copy(acc_sh.at[pl.ds(s * _RPT, _RPT)],
                        out_hbm.at[c, pl.ds(s * _RPT, _RPT)])
        if with_counts:
            pltpu.sync_copy(cnt_sh.at[pl.ds(s * _RPT, _RPT)],
                            cnt_hbm.at[c, pl.ds(s * _RPT, _RPT)])

    zero = jnp.zeros((_RPT, d), jnp.float32)
    zero16 = jnp.zeros((_RPT, 16), jnp.float32)
    ones = jnp.ones((_CHUNK, 16), jnp.float32)
    fn = pl.kernel(
        body, out_type=out_type, mesh=mesh, scratch_types=scratch,
        compiler_params=pltpu.CompilerParams(use_tc_tiling_on_sc=False))
    return fn(y, src, dst, zero, zero16, ones)


def _lin2_body(h_ref, wl_ref, wr_ref, b_ref, y_ref, z_ref):
    h = h_ref[...]
    y_ref[...] = jnp.dot(h, wl_ref[...], preferred_element_type=jnp.float32)
    z_ref[...] = (jnp.dot(h, wr_ref[...], preferred_element_type=jnp.float32)
                  + b_ref[...])


def _tc_lin2(h, wl, wr, b):
    din, dout = wl.shape
    return pl.pallas_call(
        _lin2_body,
        grid=(_GRID,),
        in_specs=[
            pl.BlockSpec((_ROW_BLK, din), lambda i: (i, 0)),
            pl.BlockSpec((din, dout), lambda i: (0, 0)),
            pl.BlockSpec((din, dout), lambda i: (0, 0)),
            pl.BlockSpec((1, dout), lambda i: (0, 0)),
        ],
        out_specs=[
            pl.BlockSpec((_ROW_BLK, dout), lambda i: (i, 0)),
            pl.BlockSpec((_ROW_BLK, dout), lambda i: (i, 0)),
        ],
        out_shape=[
            jax.ShapeDtypeStruct((_N, dout), jnp.float32),
            jax.ShapeDtypeStruct((_N, dout), jnp.float32),
        ],
    )(h, wl, wr, b.reshape(1, dout))


def _combine_h(p_ref, cnt_ref, z_ref):
    cnt = cnt_ref[0, :, 0:1] + cnt_ref[1, :, 0:1]
    scale = 1.0 / jnp.maximum(cnt, 1.0)
    return (p_ref[0] + p_ref[1]) * scale + z_ref[...]


def _comb_body(p_ref, cnt_ref, z_ref, wl_ref, wr_ref, b_ref, y_ref, z2_ref):
    h = jnp.maximum(_combine_h(p_ref, cnt_ref, z_ref), 0.0)
    y_ref[...] = jnp.dot(h, wl_ref[...], preferred_element_type=jnp.float32)
    z2_ref[...] = (jnp.dot(h, wr_ref[...], preferred_element_type=jnp.float32)
                   + b_ref[...])


def _tc_combine(p, cntp, z, wl, wr, b):
    dp = p.shape[-1]
    din, dout = wl.shape
    return pl.pallas_call(
        _comb_body,
        grid=(_GRID,),
        in_specs=[
            pl.BlockSpec((2, _ROW_BLK, dp), lambda i: (0, i, 0)),
            pl.BlockSpec((2, _ROW_BLK, 16), lambda i: (0, i, 0)),
            pl.BlockSpec((_ROW_BLK, dp), lambda i: (i, 0)),
            pl.BlockSpec((din, dout), lambda i: (0, 0)),
            pl.BlockSpec((din, dout), lambda i: (0, 0)),
            pl.BlockSpec((1, dout), lambda i: (0, 0)),
        ],
        out_specs=[
            pl.BlockSpec((_ROW_BLK, dout), lambda i: (i, 0)),
            pl.BlockSpec((_ROW_BLK, dout), lambda i: (i, 0)),
        ],
        out_shape=[
            jax.ShapeDtypeStruct((_N, dout), jnp.float32),
            jax.ShapeDtypeStruct((_N, dout), jnp.float32),
        ],
    )(p, cntp, z, wl, wr, b.reshape(1, dout))


def _final_body(p_ref, cnt_ref, z_ref, o_ref):
    h = _combine_h(p_ref, cnt_ref, z_ref)
    m = jnp.max(h, axis=1, keepdims=True)
    e = jnp.exp(h - m)
    o_ref[...] = h - (jnp.log(jnp.sum(e, axis=1, keepdims=True)) + m)


def _tc_final(p, cntp, z):
    dp = p.shape[-1]
    return pl.pallas_call(
        _final_body,
        grid=(_GRID,),
        in_specs=[
            pl.BlockSpec((2, _ROW_BLK, dp), lambda i: (0, i, 0)),
            pl.BlockSpec((2, _ROW_BLK, 16), lambda i: (0, i, 0)),
            pl.BlockSpec((_ROW_BLK, dp), lambda i: (i, 0)),
        ],
        out_specs=pl.BlockSpec((_ROW_BLK, dp), lambda i: (i, 0)),
        out_shape=jax.ShapeDtypeStruct((_N, dp), jnp.float32),
    )(p, cntp, z)


def kernel(x, edge_index, Wl1, bl1, Wr1, Wl2, bl2, Wr2,
           Wl3, bl3, Wr3, Wl4, bl4, Wr4):
    src = edge_index[0]
    dst = edge_index[1]

    y1, z1 = _tc_lin2(x, Wl1, Wr1, bl1)
    p1, cp = _sc_agg_call(y1, src, dst, 128, with_counts=True)
    p1, cp = p1[:, :_N], cp[:, :_N]

    y2, z2 = _tc_combine(p1, cp, z1, Wl2, Wr2, bl2)
    p2 = _sc_agg_call(y2, src, dst, 128, with_counts=False)[0][:, :_N]

    y3, z3 = _tc_combine(p2, cp, z2, Wl3, Wr3, bl3)
    p3 = _sc_agg_call(y3, src, dst, 64, with_counts=False)[0][:, :_N]

    y4, z4 = _tc_combine(p3, cp, z3, Wl4, Wr4, bl4)
    p4 = _sc_agg_call(y4, src, dst, 64, with_counts=False)[0][:, :_N]

    return _tc_final(p4, cp, z4)
